# baseline (speedup 1.0000x reference)
import jax
import jax.numpy as jnp
from jax import lax
from jax.experimental import pallas as pl
from jax.experimental.pallas import tpu as pltpu
from jax.experimental.pallas import tpu_sc as plsc

N = 8192
D = 128
C = 100
CPP = 128
NC = 2
NS = 16
NW = NC * NS
RPT = N // NW
LPR = 128
CLS_B = CPP // NS


def _rsqrt_nr(x):
  bits = lax.bitcast_convert_type(x, jnp.int32)
  y = lax.bitcast_convert_type(
      jnp.int32(0x5F3759DF) - (bits >> 1), jnp.float32)
  for _ in range(3):
    t = x * y
    u = t * y
    y = y * (1.5 - 0.5 * u)
  return y


def _acc_body(feat_hbm, lab_hbm, ps_hbm, pq_hbm, pc_hbm,
              fv, lab_v, ones_v, dump, acc_s, acc_q, acc_c):
  s = lax.axis_index("s")
  c = lax.axis_index("c")

  zeros16 = jnp.zeros((16,), jnp.float32)
  ones16 = jnp.full((16,), 1.0, jnp.float32)

  @pl.when(s == 0)
  def _init():
    def zf(i, cy):
      for k in range(D // 16):
        fv[i, pl.ds(k * 16, 16)] = zeros16
      return cy
    lax.fori_loop(0, CPP, zf, 0)
    pltpu.sync_copy(fv.at[pl.ds(0, CPP)], acc_s)
    pltpu.sync_copy(fv.at[pl.ds(0, CPP)], acc_q)
    pltpu.sync_copy(fv.at[pl.ds(0, CPP)], acc_c)

  def fill_ones(i, cy):
    for k in range(D // 16):
      ones_v[i, pl.ds(k * 16, 16)] = ones16
    return cy
  lax.fori_loop(0, LPR, fill_ones, 0)

  lab_row0 = c * (N // NC // LPR) + s * (RPT // LPR)
  pltpu.sync_copy(lab_hbm.at[pl.ds(lab_row0, RPT // LPR)], lab_v)
  pltpu.sync_copy(feat_hbm.at[pl.ds(c * (N // NC) + s * RPT, RPT)], fv)

  plsc.subcore_barrier()

  for j in range(RPT // LPR):
    pltpu.sync_copy(fv.at[pl.ds(j * LPR, LPR)], acc_s.at[lab_v.at[j]],
                    add=True)

  def sqr(i, cy):
    for k in range(D // 16):
      v = fv[i, pl.ds(k * 16, 16)]
      fv[i, pl.ds(k * 16, 16)] = v * v
    return cy
  lax.fori_loop(0, RPT, sqr, 0)

  for j in range(RPT // LPR):
    pltpu.sync_copy(fv.at[pl.ds(j * LPR, LPR)], acc_q.at[lab_v.at[j]],
                    add=True)

  for j in range(RPT // LPR):
    pltpu.sync_copy(ones_v, acc_c.at[lab_v.at[j]], add=True)

  plsc.subcore_barrier()

  pltpu.sync_copy(acc_s.at[pl.ds(s * CLS_B, CLS_B)], dump)
  pltpu.sync_copy(dump, ps_hbm.at[c, pl.ds(s * CLS_B, CLS_B)])
  pltpu.sync_copy(acc_q.at[pl.ds(s * CLS_B, CLS_B)], dump)
  pltpu.sync_copy(dump, pq_hbm.at[c, pl.ds(s * CLS_B, CLS_B)])
  pltpu.sync_copy(acc_c.at[pl.ds(s * CLS_B, CLS_B)], dump)
  pltpu.sync_copy(dump, pc_hbm.at[c, pl.ds(s * CLS_B, CLS_B)])


def _fin_body(ps_hbm, pq_hbm, pc_hbm, lab_hbm, out_hbm,
              gbuf, lab_v, a8, b8, qa8, qb8, ca8, cb8, stdv, std_s):
  s = lax.axis_index("s")
  c = lax.axis_index("c")
  gw = c * NS + s
  cls0 = s * CLS_B

  pltpu.sync_copy(ps_hbm.at[0, pl.ds(cls0, CLS_B)], a8)
  pltpu.sync_copy(ps_hbm.at[1, pl.ds(cls0, CLS_B)], b8)
  pltpu.sync_copy(pq_hbm.at[0, pl.ds(cls0, CLS_B)], qa8)
  pltpu.sync_copy(pq_hbm.at[1, pl.ds(cls0, CLS_B)], qb8)
  pltpu.sync_copy(pc_hbm.at[0, pl.ds(cls0, CLS_B)], ca8)
  pltpu.sync_copy(pc_hbm.at[1, pl.ds(cls0, CLS_B)], cb8)

  for r in range(CLS_B):
    cnt = ca8[r, pl.ds(0, 16)] + cb8[r, pl.ds(0, 16)]
    inv = 1.0 / jnp.maximum(cnt, 1.0)
    for k in range(D // 16):
      sv = a8[r, pl.ds(k * 16, 16)] + b8[r, pl.ds(k * 16, 16)]
      qv = qa8[r, pl.ds(k * 16, 16)] + qb8[r, pl.ds(k * 16, 16)]
      mean = sv * inv
      var = qv * inv - mean * mean
      var = jnp.maximum(var, 1e-30)
      stdv[r, pl.ds(k * 16, 16)] = var * _rsqrt_nr(var)

  pltpu.sync_copy(stdv, std_s.at[pl.ds(cls0, CLS_B)])

  plsc.subcore_barrier()

  for j in range(RPT // LPR):
    pltpu.sync_copy(lab_hbm.at[pl.ds(gw * (RPT // LPR) + j, 1)],
                    lab_v.at[pl.ds(j, 1)])
  for j in range(RPT // LPR):
    pltpu.sync_copy(std_s.at[lab_v.at[j]], gbuf.at[pl.ds(j * LPR, LPR)])
  pltpu.sync_copy(gbuf, out_hbm.at[pl.ds(gw * RPT, RPT)])


_mesh = plsc.VectorSubcoreMesh(
    core_axis_name="c", subcore_axis_name="s",
    num_cores=NC, num_subcores=NS)

_acc_call = pl.kernel(
    _acc_body,
    out_type=[
        jax.ShapeDtypeStruct((NC, CPP, D), jnp.float32),
        jax.ShapeDtypeStruct((NC, CPP, D), jnp.float32),
        jax.ShapeDtypeStruct((NC, CPP, D), jnp.float32),
    ],
    mesh=_mesh,
    scratch_types=[
        pltpu.VMEM((RPT, D), jnp.float32),
        pltpu.VMEM((RPT // LPR, LPR), jnp.int32),
        pltpu.VMEM((LPR, D), jnp.float32),
        pltpu.VMEM((CLS_B, D), jnp.float32),
        pltpu.VMEM_SHARED((CPP, D), jnp.float32),
        pltpu.VMEM_SHARED((CPP, D), jnp.float32),
        pltpu.VMEM_SHARED((CPP, D), jnp.float32),
    ],
)

_fin_call = pl.kernel(
    _fin_body,
    out_type=jax.ShapeDtypeStruct((N, D), jnp.float32),
    mesh=_mesh,
    scratch_types=[
        pltpu.VMEM((RPT, D), jnp.float32),
        pltpu.VMEM((RPT // LPR, LPR), jnp.int32),
        pltpu.VMEM((CLS_B, D), jnp.float32),
        pltpu.VMEM((CLS_B, D), jnp.float32),
        pltpu.VMEM((CLS_B, D), jnp.float32),
        pltpu.VMEM((CLS_B, D), jnp.float32),
        pltpu.VMEM((CLS_B, D), jnp.float32),
        pltpu.VMEM((CLS_B, D), jnp.float32),
        pltpu.VMEM((CLS_B, D), jnp.float32),
        pltpu.VMEM_SHARED((CPP, D), jnp.float32),
    ],
)


@jax.jit
def kernel(features, labels):
  lab2 = labels.astype(jnp.int32).reshape(N // LPR, LPR)
  ps, pq, pc = _acc_call(features, lab2)
  return _fin_call(ps, pq, pc, lab2)


# speedup vs baseline: 1.0947x; 1.0947x over previous
import jax
import jax.numpy as jnp
from jax import lax
from jax.experimental import pallas as pl
from jax.experimental.pallas import tpu as pltpu
from jax.experimental.pallas import tpu_sc as plsc

N = 8192
D = 128
C = 100
CP = 112
NC = 2
NS = 16
NW = NC * NS
ROWS_ACC = N // NS
CHUNK = ROWS_ACC // 2
ROWS_OUT = N // NW
CLS_PER_TILE = CP // NS
LPR = 128


def _rsqrt_nr(x):
  bits = lax.bitcast_convert_type(x, jnp.int32)
  y = lax.bitcast_convert_type(
      jnp.int32(0x5F3759DF) - (bits >> 1), jnp.float32)
  for _ in range(3):
    t = x * y
    u = t * y
    y = y * (1.5 - 0.5 * u)
  return y


def _body(feat_hbm, lab_hbm, out_hbm,
          fva, fvb, lab_v, ones_v, srow, qrow, ctmp, stdv,
          sem_a, sem_b, sem_c, sem_g,
          acc_s, acc_q, acc_c, std_s):
  s = lax.axis_index("s")
  c = lax.axis_index("c")
  gw = c * NS + s

  zeros16 = jnp.zeros((16,), jnp.float32)
  ones16 = jnp.full((16,), 1.0, jnp.float32)

  dg = pltpu.async_copy(lab_hbm.at[pl.ds(gw * 2, 2)],
                        lab_v.at[pl.ds(4, 2)], sem_g)
  pltpu.sync_copy(lab_hbm.at[pl.ds(s * 4, 4)], lab_v.at[pl.ds(0, 4)])

  @pl.when(s == 0)
  def _init():
    def zf(i, cy):
      for k in range(D // 16):
        fva[i, pl.ds(k * 16, 16)] = zeros16
      return cy
    lax.fori_loop(0, CP, zf, 0)
    pltpu.sync_copy(fva.at[pl.ds(0, CP)], acc_s)
    pltpu.sync_copy(fva.at[pl.ds(0, CP)], acc_q)
    pltpu.sync_copy(fva.at[pl.ds(0, CP)], acc_c)

  def fill_ones(i, cy):
    for k in range(D // 16):
      ones_v[i, pl.ds(k * 16, 16)] = ones16
    return cy
  lax.fori_loop(0, LPR, fill_ones, 0)

  l0 = pltpu.async_copy(feat_hbm.at[pl.ds(s * ROWS_ACC, CHUNK)], fva, sem_a)
  l1 = pltpu.async_copy(feat_hbm.at[pl.ds(s * ROWS_ACC + CHUNK, CHUNK)],
                        fvb, sem_b)

  plsc.subcore_barrier()

  cns = [pltpu.async_copy(ones_v, acc_c.at[lab_v.at[j]], sem_c, add=True)
         for j in range(4)]

  def sqr(buf):
    def go(i, cy):
      for k in range(D // 16):
        v = buf[i, pl.ds(k * 16, 16)]
        buf[i, pl.ds(k * 16, 16)] = v * v
      return cy
    lax.fori_loop(0, CHUNK, go, 0)

  l0.wait()
  for j in range(2):
    pltpu.sync_copy(fva.at[pl.ds(j * LPR, LPR)], acc_s.at[lab_v.at[j]],
                    add=True)
  sqr(fva)
  q0 = [pltpu.async_copy(fva.at[pl.ds(j * LPR, LPR)], acc_q.at[lab_v.at[j]],
                         sem_a, add=True) for j in range(2)]

  l1.wait()
  for j in range(2):
    pltpu.sync_copy(fvb.at[pl.ds(j * LPR, LPR)], acc_s.at[lab_v.at[j + 2]],
                    add=True)
  sqr(fvb)
  q1 = [pltpu.async_copy(fvb.at[pl.ds(j * LPR, LPR)],
                         acc_q.at[lab_v.at[j + 2]], sem_b, add=True)
        for j in range(2)]

  for d in q0 + q1 + cns:
    d.wait()

  plsc.subcore_barrier()

  cls0 = s * CLS_PER_TILE
  f0 = pltpu.async_copy(acc_s.at[pl.ds(cls0, CLS_PER_TILE)], srow, sem_a)
  f1 = pltpu.async_copy(acc_q.at[pl.ds(cls0, CLS_PER_TILE)], qrow, sem_b)
  f2 = pltpu.async_copy(acc_c.at[pl.ds(cls0, CLS_PER_TILE)], ctmp, sem_c)
  f0.wait(); f1.wait(); f2.wait()

  for r in range(CLS_PER_TILE):
    cnt = ctmp[r, pl.ds(0, 16)]
    inv = 1.0 / jnp.maximum(cnt, 1.0)
    for k in range(D // 16):
      sv = srow[r, pl.ds(k * 16, 16)]
      qv = qrow[r, pl.ds(k * 16, 16)]
      mean = sv * inv
      var = qv * inv - mean * mean
      var = jnp.maximum(var, 1e-30)
      stdv[r, pl.ds(k * 16, 16)] = var * _rsqrt_nr(var)

  pltpu.sync_copy(stdv, std_s.at[pl.ds(cls0, CLS_PER_TILE)])

  plsc.subcore_barrier()

  dg.wait()
  g0 = pltpu.async_copy(std_s.at[lab_v.at[4]], fva.at[pl.ds(0, LPR)], sem_a)
  g1 = pltpu.async_copy(std_s.at[lab_v.at[5]], fva.at[pl.ds(LPR, LPR)], sem_b)
  g0.wait()
  g1.wait()
  pltpu.sync_copy(fva.at[pl.ds(0, ROWS_OUT)],
                  out_hbm.at[pl.ds(gw * ROWS_OUT, ROWS_OUT)])


_sc_call = pl.kernel(
    _body,
    out_type=jax.ShapeDtypeStruct((N, D), jnp.float32),
    mesh=plsc.VectorSubcoreMesh(
        core_axis_name="c", subcore_axis_name="s",
        num_cores=NC, num_subcores=NS),
    scratch_types=[
        pltpu.VMEM((CHUNK, D), jnp.float32),
        pltpu.VMEM((CHUNK, D), jnp.float32),
        pltpu.VMEM((6, LPR), jnp.int32),
        pltpu.VMEM((LPR, D), jnp.float32),
        pltpu.VMEM((CLS_PER_TILE, D), jnp.float32),
        pltpu.VMEM((CLS_PER_TILE, D), jnp.float32),
        pltpu.VMEM((CLS_PER_TILE, D), jnp.float32),
        pltpu.VMEM((CLS_PER_TILE, D), jnp.float32),
        pltpu.SemaphoreType.DMA,
        pltpu.SemaphoreType.DMA,
        pltpu.SemaphoreType.DMA,
        pltpu.SemaphoreType.DMA,
        pltpu.VMEM_SHARED((CP, D), jnp.float32),
        pltpu.VMEM_SHARED((CP, D), jnp.float32),
        pltpu.VMEM_SHARED((CP, D), jnp.float32),
        pltpu.VMEM_SHARED((CP, D), jnp.float32),
    ],
)


@jax.jit
def kernel(features, labels):
  lab2 = labels.astype(jnp.int32).reshape(N // LPR, LPR)
  return _sc_call(features, lab2)


# speedup vs baseline: 1.1215x; 1.0244x over previous
import jax
import jax.numpy as jnp
from jax import lax
from jax.experimental import pallas as pl
from jax.experimental.pallas import tpu as pltpu
from jax.experimental.pallas import tpu_sc as plsc

N = 8192
D = 128
C = 100
CP = 112
NC = 2
NS = 16
NW = NC * NS
ROWS_ACC = N // NS
CHUNK = ROWS_ACC // 2
ROWS_OUT = N // NW
CLS_PER_TILE = CP // NS
LPR = 128


def _rsqrt_nr(x):
  bits = lax.bitcast_convert_type(x, jnp.int32)
  y = lax.bitcast_convert_type(
      jnp.int32(0x5F3759DF) - (bits >> 1), jnp.float32)
  for _ in range(3):
    t = x * y
    u = t * y
    y = y * (1.5 - 0.5 * u)
  return y


def _body(feat_hbm, lab_hbm, out_hbm,
          fva, fvb, sqa, sqb, lab_v, ones_v, srow, qrow, ctmp, stdv,
          sem_a, sem_b, sem_c, sem_g, sem_s, sem_q,
          acc_s, acc_q, acc_c, std_s):
  s = lax.axis_index("s")
  c = lax.axis_index("c")
  gw = c * NS + s

  zeros16 = jnp.zeros((16,), jnp.float32)
  ones16 = jnp.full((16,), 1.0, jnp.float32)

  dg = pltpu.async_copy(lab_hbm.at[pl.ds(gw * 2, 2)],
                        lab_v.at[pl.ds(4, 2)], sem_g)
  pltpu.sync_copy(lab_hbm.at[pl.ds(s * 4, 4)], lab_v.at[pl.ds(0, 4)])

  @pl.when(s == 0)
  def _init():
    def zf(i, cy):
      for k in range(D // 16):
        sqa[i, pl.ds(k * 16, 16)] = zeros16
      return cy
    lax.fori_loop(0, CP, zf, 0)
    pltpu.sync_copy(sqa.at[pl.ds(0, CP)], acc_s)
    pltpu.sync_copy(sqa.at[pl.ds(0, CP)], acc_q)
    pltpu.sync_copy(sqa.at[pl.ds(0, CP)], acc_c)

  def fill_ones(i, cy):
    for k in range(D // 16):
      ones_v[i, pl.ds(k * 16, 16)] = ones16
    return cy
  lax.fori_loop(0, LPR, fill_ones, 0)

  fbuf = (fva, fvb)
  qbuf = (sqa, sqb)
  lsem = (sem_a, sem_b)
  loads = [None] * 4
  loads[0] = pltpu.async_copy(feat_hbm.at[pl.ds(s * ROWS_ACC, LPR)],
                              fva, sem_a)
  loads[1] = pltpu.async_copy(feat_hbm.at[pl.ds(s * ROWS_ACC + LPR, LPR)],
                              fvb, sem_b)

  plsc.subcore_barrier()

  cns = [pltpu.async_copy(ones_v, acc_c.at[lab_v.at[j]], sem_c, add=True)
         for j in range(4)]

  def sqr(src, dst):
    def go(i, cy):
      for k in range(D // 16):
        v = src[i, pl.ds(k * 16, 16)]
        dst[i, pl.ds(k * 16, 16)] = v * v
      return cy
    lax.fori_loop(0, LPR, go, 0)

  ssc = [None] * 4
  qsc = [None] * 4
  for t in range(4):
    b = t % 2
    loads[t].wait()
    ssc[t] = pltpu.async_copy(fbuf[b], acc_s.at[lab_v.at[t]], sem_s,
                              add=True)
    if t >= 2:
      qsc[t - 2].wait()
    sqr(fbuf[b], qbuf[b])
    qsc[t] = pltpu.async_copy(qbuf[b], acc_q.at[lab_v.at[t]], sem_q,
                              add=True)
    if t < 2:
      ssc[t].wait()
      loads[t + 2] = pltpu.async_copy(
          feat_hbm.at[pl.ds(s * ROWS_ACC + (t + 2) * LPR, LPR)],
          fbuf[b], lsem[b])
    else:
      ssc[t].wait()

  for d in qsc[2:] + cns:
    d.wait()

  plsc.subcore_barrier()

  cls0 = s * CLS_PER_TILE
  f0 = pltpu.async_copy(acc_s.at[pl.ds(cls0, CLS_PER_TILE)], srow, sem_a)
  f1 = pltpu.async_copy(acc_q.at[pl.ds(cls0, CLS_PER_TILE)], qrow, sem_b)
  f2 = pltpu.async_copy(acc_c.at[pl.ds(cls0, CLS_PER_TILE)], ctmp, sem_c)
  f0.wait(); f1.wait(); f2.wait()

  for r in range(CLS_PER_TILE):
    cnt = ctmp[r, pl.ds(0, 16)]
    inv = 1.0 / jnp.maximum(cnt, 1.0)
    for k in range(D // 16):
      sv = srow[r, pl.ds(k * 16, 16)]
      qv = qrow[r, pl.ds(k * 16, 16)]
      mean = sv * inv
      var = qv * inv - mean * mean
      var = jnp.maximum(var, 1e-30)
      stdv[r, pl.ds(k * 16, 16)] = var * _rsqrt_nr(var)

  pltpu.sync_copy(stdv, std_s.at[pl.ds(cls0, CLS_PER_TILE)])

  plsc.subcore_barrier()

  dg.wait()
  g0 = pltpu.async_copy(std_s.at[lab_v.at[4]], fva, sem_a)
  g1 = pltpu.async_copy(std_s.at[lab_v.at[5]], fvb, sem_b)
  g0.wait()
  w0 = pltpu.async_copy(fva, out_hbm.at[pl.ds(gw * ROWS_OUT, LPR)], sem_s)
  g1.wait()
  w1 = pltpu.async_copy(fvb, out_hbm.at[pl.ds(gw * ROWS_OUT + LPR, LPR)],
                        sem_q)
  w0.wait()
  w1.wait()


_sc_call = pl.kernel(
    _body,
    out_type=jax.ShapeDtypeStruct((N, D), jnp.float32),
    mesh=plsc.VectorSubcoreMesh(
        core_axis_name="c", subcore_axis_name="s",
        num_cores=NC, num_subcores=NS),
    scratch_types=[
        pltpu.VMEM((LPR, D), jnp.float32),
        pltpu.VMEM((LPR, D), jnp.float32),
        pltpu.VMEM((LPR, D), jnp.float32),
        pltpu.VMEM((LPR, D), jnp.float32),
        pltpu.VMEM((6, LPR), jnp.int32),
        pltpu.VMEM((LPR, D), jnp.float32),
        pltpu.VMEM((CLS_PER_TILE, D), jnp.float32),
        pltpu.VMEM((CLS_PER_TILE, D), jnp.float32),
        pltpu.VMEM((CLS_PER_TILE, D), jnp.float32),
        pltpu.VMEM((CLS_PER_TILE, D), jnp.float32),
        pltpu.SemaphoreType.DMA,
        pltpu.SemaphoreType.DMA,
        pltpu.SemaphoreType.DMA,
        pltpu.SemaphoreType.DMA,
        pltpu.SemaphoreType.DMA,
        pltpu.SemaphoreType.DMA,
        pltpu.VMEM_SHARED((CP, D), jnp.float32),
        pltpu.VMEM_SHARED((CP, D), jnp.float32),
        pltpu.VMEM_SHARED((CP, D), jnp.float32),
        pltpu.VMEM_SHARED((CP, D), jnp.float32),
    ],
)


@jax.jit
def kernel(features, labels):
  lab2 = labels.astype(jnp.int32).reshape(N // LPR, LPR)
  return _sc_call(features, lab2)
